# Initial kernel scaffold; baseline (speedup 1.0000x reference)
import functools

import jax
import jax.numpy as jnp
from jax.experimental import pallas as pl
from jax.experimental.pallas import tpu as pltpu
from jax.experimental.pallas import tpu_sc as plsc

_WINDOW = 128


@functools.partial(jax.jit, static_argnums=(2, 3))
def _sc_gather(table, idx_flat, n_idx, embed):
  mesh = plsc.VectorSubcoreMesh(core_axis_name="core",
                                subcore_axis_name="subcore")

  @functools.partial(
      pl.kernel,
      out_type=jax.ShapeDtypeStruct((n_idx, embed), table.dtype),
      mesh=mesh,
  )
  def gather_kernel(table_hbm, idx_hbm, out_hbm):
    def body(idx_vmem, out_vmem):
      pltpu.sync_copy(table_hbm.at[idx_vmem.at[0]], out_vmem)

    pltpu.emit_pipeline(
        body,
        grid=(n_idx // _WINDOW,),
        in_specs=[pl.BlockSpec((1, _WINDOW), index_map=lambda i: (0, i))],
        out_specs=[pl.BlockSpec((_WINDOW, embed), index_map=lambda i: (i, 0))],
        core_axis_name=("core", "subcore"),
        dimension_semantics=(pltpu.PARALLEL,),
    )(idx_hbm, out_hbm)

  return gather_kernel(table, idx_flat)


def kernel(table, type_index):
  batch, hist = type_index.shape
  embed = table.shape[1]
  n_idx = batch * hist
  idx_flat = type_index.reshape(1, n_idx)
  out = _sc_gather(table, idx_flat, n_idx, embed)
  return out.reshape(batch, hist, embed)


# speedup vs baseline: 1.7451x; 1.7451x over previous
import functools

import jax
import jax.numpy as jnp
from jax.experimental import pallas as pl
from jax.experimental.pallas import tpu as pltpu
from jax.experimental.pallas import tpu_sc as plsc

_WINDOW = 128


@functools.partial(jax.jit, static_argnums=(2, 3))
def _sc_gather(table, idx_flat, n_idx, embed):
  mesh = plsc.VectorSubcoreMesh(core_axis_name="core",
                                subcore_axis_name="subcore")

  @functools.partial(
      pl.kernel,
      out_type=jax.ShapeDtypeStruct((n_idx, embed), table.dtype),
      mesh=mesh,
      compiler_params=pltpu.CompilerParams(use_tc_tiling_on_sc=False),
  )
  def gather_kernel(table_hbm, idx_hbm, out_hbm):
    def body(idx_vmem, out_vmem):
      pltpu.sync_copy(table_hbm.at[idx_vmem.at[0]], out_vmem)

    pltpu.emit_pipeline(
        body,
        grid=(n_idx // _WINDOW,),
        in_specs=[pl.BlockSpec((1, _WINDOW), index_map=lambda i: (0, i))],
        out_specs=[pl.BlockSpec((_WINDOW, embed), index_map=lambda i: (i, 0))],
        core_axis_name=("core", "subcore"),
        dimension_semantics=(pltpu.PARALLEL,),
    )(idx_hbm, out_hbm)

  return gather_kernel(table, idx_flat)


def kernel(table, type_index):
  batch, hist = type_index.shape
  embed = table.shape[1]
  n_idx = batch * hist
  idx_flat = type_index.reshape(1, n_idx)
  out = _sc_gather(table, idx_flat, n_idx, embed)
  return out.reshape(batch, hist, embed)


# speedup vs baseline: 1.8727x; 1.0731x over previous
import functools

import jax
import jax.numpy as jnp
from jax.experimental import pallas as pl
from jax.experimental.pallas import tpu as pltpu
from jax.experimental.pallas import tpu_sc as plsc

_WINDOW = 512


@functools.partial(jax.jit, static_argnums=(2, 3))
def _sc_gather(table, idx_flat, n_idx, embed):
  mesh = plsc.VectorSubcoreMesh(core_axis_name="core",
                                subcore_axis_name="subcore")

  @functools.partial(
      pl.kernel,
      out_type=jax.ShapeDtypeStruct((n_idx, embed), table.dtype),
      mesh=mesh,
      compiler_params=pltpu.CompilerParams(use_tc_tiling_on_sc=False),
  )
  def gather_kernel(table_hbm, idx_hbm, out_hbm):
    def body(idx_vmem, out_vmem):
      pltpu.sync_copy(table_hbm.at[idx_vmem.at[0]], out_vmem)

    pltpu.emit_pipeline(
        body,
        grid=(n_idx // _WINDOW,),
        in_specs=[pl.BlockSpec((1, _WINDOW), index_map=lambda i: (0, i))],
        out_specs=[pl.BlockSpec((_WINDOW, embed), index_map=lambda i: (i, 0))],
        core_axis_name=("core", "subcore"),
        dimension_semantics=(pltpu.PARALLEL,),
    )(idx_hbm, out_hbm)

  return gather_kernel(table, idx_flat)


def kernel(table, type_index):
  batch, hist = type_index.shape
  embed = table.shape[1]
  n_idx = batch * hist
  idx_flat = type_index.reshape(1, n_idx)
  out = _sc_gather(table, idx_flat, n_idx, embed)
  return out.reshape(batch, hist, embed)
